# baseline (speedup 1.0000x reference)
import functools

import jax
import jax.numpy as jnp
from jax import lax
from jax.experimental import pallas as pl
from jax.experimental.pallas import tpu as pltpu
from jax.experimental.pallas import tpu_sc as plsc

B, S, D = 16, 128, 256
M_VALID = 2
KT = 8
T_IMG, KEPT_IMG = 196, 50
T_NLP, KEPT_NLP = 256, 65

ROWS_T = B * S * 3
ROWS_IMG = B * KEPT_IMG
ROWS_NLP = B * KEPT_NLP
BASE_IMG = ROWS_T
BASE_NLP = ROWS_T + ROWS_IMG
MASK_ROW = ROWS_T + ROWS_IMG + ROWS_NLP

NW = 32
PAIRS_PER_W = (B * S) // NW
TROWS_PER_W = PAIRS_PER_W * (KT + 1)
T_CHUNK = 96

IMG_PAD_T = 208
IMG_PAD_M = 64
NLP_PAD_M = 80


def _iota16():
    return lax.iota(jnp.int32, 16)


def _compute_temporal_gidx(wid, tix, gidxt):
    r0 = wid * PAIRS_PER_W

    def body(c, carry):
        q = c * 16 + _iota16()
        j = lax.shift_right_logical(q * 7282, 16)
        k = q - j * (KT + 1)
        pos = jnp.maximum(j * KT + k - 1, 0)
        raw = plsc.load_gather(tix, [pos])
        gr3 = (r0 + j) * 3
        src = jnp.where(k == 0, gr3,
                        jnp.where(raw < M_VALID, gr3 + 1 + raw, MASK_ROW))
        gidxt[pl.ds(c * 16, 16)] = src
        return carry

    lax.fori_loop(0, TROWS_PER_W // 16, body, 0)


def _compute_static_gidx(b, t_len, kept, base, six, mskv, gidxs, n_chunks):
    def body(c, carry):
        p = c * 16 + _iota16()
        pos = jnp.clip(p - 1, 0, t_len - 1)
        raw = plsc.load_gather(six, [pos])
        srow = jnp.where(p == 0, 0, 1 + raw)
        mg = jnp.minimum(srow, kept - 1)
        mval = plsc.load_gather(mskv, [mg])
        valid = (srow <= kept - 1) & (mval == 1)
        src = jnp.where(valid, base + b * kept + srow, MASK_ROW)
        gidxs[pl.ds(c * 16, 16)] = src
        return carry

    lax.fori_loop(0, n_chunks, body, 0)


def _gather_rows(table, gidx, rows, sem, out, out_base, chunks):
    for off, n in chunks:
        pltpu.async_copy(
            table.at[gidx.at[pl.ds(off, n)]], rows.at[pl.ds(0, n)], sem).wait()
        pltpu.sync_copy(rows.at[pl.ds(0, n)], out.at[pl.ds(out_base + off, n)])


def _sc_body(table, t_idx, img_idx, nlp_idx, img_mask, nlp_mask,
             t_out, img_out, nlp_out, tix, gidxt, six, mskv, gidxs, rows, sem):
    wid = lax.axis_index("s") * 2 + lax.axis_index("c")

    pltpu.sync_copy(t_idx.at[pl.ds(wid * PAIRS_PER_W * KT, PAIRS_PER_W * KT)],
                    tix)
    _compute_temporal_gidx(wid, tix, gidxt)
    t_chunks = [(h * T_CHUNK, T_CHUNK)
                for h in range(TROWS_PER_W // T_CHUNK)]
    _gather_rows(table, gidxt, rows, sem, t_out, wid * TROWS_PER_W, t_chunks)

    b = wid // 2
    m = wid - 2 * b

    @pl.when(m == 0)
    def _img():
        pltpu.sync_copy(img_idx.at[b], six.at[pl.ds(0, IMG_PAD_T)])
        pltpu.sync_copy(img_mask.at[b], mskv.at[pl.ds(0, IMG_PAD_M)])
        _compute_static_gidx(b, T_IMG, KEPT_IMG, BASE_IMG, six, mskv, gidxs,
                             (T_IMG + 1 + 15) // 16)
        _gather_rows(table, gidxs, rows, sem, img_out, b * (T_IMG + 1),
                     [(0, 112), (112, 85)])

    @pl.when(m == 1)
    def _nlp():
        pltpu.sync_copy(nlp_idx.at[b], six)
        pltpu.sync_copy(nlp_mask.at[b], mskv)
        _compute_static_gidx(b, T_NLP, KEPT_NLP, BASE_NLP, six, mskv, gidxs,
                             (T_NLP + 1 + 15) // 16)
        _gather_rows(table, gidxs, rows, sem, nlp_out, b * (T_NLP + 1),
                     [(0, 112), (112, 112), (224, 33)])


@jax.jit
def _revert_sc(table, t_idx, img_idx, nlp_idx, img_mask, nlp_mask):
    mesh = plsc.VectorSubcoreMesh(core_axis_name="c", subcore_axis_name="s")
    f = pl.kernel(
        _sc_body,
        out_type=[
            jax.ShapeDtypeStruct((B * S * (KT + 1), D), jnp.float32),
            jax.ShapeDtypeStruct((B * (T_IMG + 1), D), jnp.float32),
            jax.ShapeDtypeStruct((B * (T_NLP + 1), D), jnp.float32),
        ],
        mesh=mesh,
        compiler_params=pltpu.CompilerParams(needs_layout_passes=False,
                                             use_tc_tiling_on_sc=False),
        scratch_types=[
            pltpu.VMEM((PAIRS_PER_W * KT,), jnp.int32),
            pltpu.VMEM((TROWS_PER_W,), jnp.int32),
            pltpu.VMEM((T_NLP,), jnp.int32),
            pltpu.VMEM((NLP_PAD_M,), jnp.int32),
            pltpu.VMEM((272,), jnp.int32),
            pltpu.VMEM((112, D), jnp.float32),
            pltpu.SemaphoreType.DMA,
        ],
    )
    return f(table, t_idx, img_idx, nlp_idx, img_mask, nlp_mask)


def kernel(temporal, img, nlp, temporal_revert_idx, img_revert_idx,
           nlp_revert_idx, img_remain_padding_mask, nlp_remain_padding_mask,
           mask_token):
    table = jnp.concatenate(
        [temporal.reshape(ROWS_T, D), img.reshape(ROWS_IMG, D),
         nlp.reshape(ROWS_NLP, D), mask_token], axis=0)
    t_idx = temporal_revert_idx.reshape(B * S * KT).astype(jnp.int32)
    img_idx = jnp.pad(img_revert_idx.astype(jnp.int32),
                      ((0, 0), (0, IMG_PAD_T - T_IMG)))
    nlp_idx = nlp_revert_idx.astype(jnp.int32)
    img_mask = jnp.pad(img_remain_padding_mask.astype(jnp.int32),
                       ((0, 0), (0, IMG_PAD_M - KEPT_IMG)))
    nlp_mask = jnp.pad(nlp_remain_padding_mask.astype(jnp.int32),
                       ((0, 0), (0, NLP_PAD_M - KEPT_NLP)))
    t_out, i_out, n_out = _revert_sc(table, t_idx, img_idx, nlp_idx,
                                     img_mask, nlp_mask)
    return (t_out.reshape(B, S, KT + 1, D),
            i_out.reshape(B, T_IMG + 1, D),
            n_out.reshape(B, T_NLP + 1, D))


# speedup vs baseline: 2.3830x; 2.3830x over previous
import jax
import jax.numpy as jnp
from jax import lax
from jax.experimental import pallas as pl
from jax.experimental.pallas import tpu as pltpu
from jax.experimental.pallas import tpu_sc as plsc

B, S, D = 16, 128, 256
M_VALID = 2
KT = 8
T_IMG, KEPT_IMG = 196, 50
T_NLP, KEPT_NLP = 256, 65

NW = 32
PAIRS_PER_W = (B * S) // NW
TROWS_PER_W = PAIRS_PER_W * (KT + 1)
T_CHUNK = 96

SLAB = PAIRS_PER_W * 3
AMASK = SLAB
NMREP = 16
ABATCH = SLAB + NMREP
TREG = 280

STG_IMG = 56
STG_NLP = 72


def _iota16():
    return lax.iota(jnp.int32, 16)


def _compute_temporal_lidx(abase, tix, lidx):
    def body(c, carry):
        q = c * 16 + _iota16()
        j = lax.shift_right_logical(q * 7282, 16)
        k = q - j * (KT + 1)
        pos = jnp.maximum(j * KT + k - 1, 0)
        raw = plsc.load_gather(tix, [pos])
        mrow = AMASK + (q & (NMREP - 1))
        src = abase + jnp.where(k == 0, j * 3,
                                jnp.where(raw < M_VALID, j * 3 + 1 + raw,
                                          mrow))
        lidx[pl.ds(c * 16, 16)] = src
        return carry

    lax.fori_loop(0, TROWS_PER_W // 16, body, 0)


def _compute_static_lidx(abase, dbase, b, t_len, kept, six, mskv, lidx,
                         didx, n_chunks):
    ibase = b * t_len
    mbase = b * kept

    def body(c, carry):
        p = c * 16 + _iota16()
        pos = ibase + jnp.clip(p - 1, 0, t_len - 1)
        raw = plsc.load_gather(six, [pos])
        srow = jnp.where(p == 0, 0, 1 + raw)
        mg = jnp.minimum(srow, kept - 1)
        mval = plsc.load_gather(mskv, [mbase + mg])
        valid = (srow <= kept - 1) & (mval == 1)
        mrow = abase + AMASK + (p & (NMREP - 1))
        lidx[pl.ds(c * 16, 16)] = jnp.where(valid, dbase + srow, mrow)
        didx[pl.ds(c * 16, 16)] = b * (t_len + 1) + jnp.minimum(p, t_len)
        return carry

    lax.fori_loop(0, n_chunks, body, 0)


def _gather_rows(arena, lidx, rows, sem, sem2, out, out_base, chunks,
                 didx=None):
    nc = len(chunks)

    def start_g(i, buf):
        off, n = chunks[i]
        return pltpu.async_copy(
            arena.at[lidx.at[pl.ds(off, n)]], rows.at[buf].at[pl.ds(0, n)],
            sem)

    def start_s(i, buf):
        off, n = chunks[i]
        if didx is None:
            dst = out.at[pl.ds(out_base + off, n)]
        else:
            dst = out.at[didx.at[pl.ds(off, n)]]
        return pltpu.async_copy(rows.at[buf].at[pl.ds(0, n)], dst, sem2)

    gd = [None] * nc
    sd = [None] * nc
    gd[0] = start_g(0, 0)
    for i in range(nc):
        gd[i].wait()
        if i + 1 < nc:
            if i >= 1:
                sd[i - 1].wait()
            gd[i + 1] = start_g(i + 1, (i + 1) % 2)
        sd[i] = start_s(i, i % 2)
    if nc >= 2:
        sd[nc - 2].wait()
    sd[nc - 1].wait()


def _sc_body(t_flat, img_flat, nlp_flat, mask_rep, t_idx, img_idx, nlp_idx,
             img_mask, nlp_mask, t_out, img_out, nlp_out, table,
             tix, six, mskv, lidx, didx, rows, sem, sem2):
    wid = lax.axis_index("s") * 2 + lax.axis_index("c")
    abase = wid * TREG

    pltpu.sync_copy(t_idx.at[pl.ds(wid * PAIRS_PER_W * KT, PAIRS_PER_W * KT)],
                    tix)
    pltpu.sync_copy(t_flat.at[pl.ds(wid * SLAB, SLAB)],
                    table.at[pl.ds(abase, SLAB)])
    pltpu.sync_copy(mask_rep, table.at[pl.ds(abase + AMASK, NMREP)])
    _compute_temporal_lidx(abase, tix, lidx)
    t_chunks = [(h * T_CHUNK, T_CHUNK) for h in range(TROWS_PER_W // T_CHUNK)]
    _gather_rows(table, lidx, rows, sem, sem2, t_out, wid * TROWS_PER_W,
                 t_chunks)

    b = wid // 2
    m = wid - 2 * b

    @pl.when(m == 0)
    def _img():
        pltpu.sync_copy(img_idx, six.at[pl.ds(0, B * T_IMG)])
        pltpu.sync_copy(img_mask, mskv.at[pl.ds(0, B * KEPT_IMG)])
        r = (2 * b) & 7
        pltpu.sync_copy(img_flat.at[pl.ds(pl.multiple_of(b * KEPT_IMG - r, 8), STG_IMG)],
                        table.at[pl.ds(abase + ABATCH, STG_IMG)])
        _compute_static_lidx(abase, abase + ABATCH + r, b, T_IMG, KEPT_IMG,
                             six, mskv, lidx, didx, (T_IMG + 1 + 15) // 16)
        _gather_rows(table, lidx, rows, sem, sem2, img_out, 0,
                     [(0, 112), (112, 88)], didx=didx)

    @pl.when(m == 1)
    def _nlp():
        pltpu.sync_copy(nlp_idx, six)
        pltpu.sync_copy(nlp_mask, mskv)
        r = b & 7
        pltpu.sync_copy(nlp_flat.at[pl.ds(pl.multiple_of(b * KEPT_NLP - r, 8), STG_NLP)],
                        table.at[pl.ds(abase + ABATCH, STG_NLP)])
        _compute_static_lidx(abase, abase + ABATCH + r, b, T_NLP, KEPT_NLP,
                             six, mskv, lidx, didx, (T_NLP + 1 + 15) // 16)
        _gather_rows(table, lidx, rows, sem, sem2, nlp_out, 0,
                     [(0, 112), (112, 112), (224, 40)], didx=didx)


@jax.jit
def _revert_sc(t_flat, img_flat, nlp_flat, mask_rep, t_idx, img_idx, nlp_idx,
               img_mask, nlp_mask):
    mesh = plsc.VectorSubcoreMesh(core_axis_name="c", subcore_axis_name="s")
    f = pl.kernel(
        _sc_body,
        out_type=[
            jax.ShapeDtypeStruct((B * S * (KT + 1), D), jnp.float32),
            jax.ShapeDtypeStruct((B * (T_IMG + 1), D), jnp.float32),
            jax.ShapeDtypeStruct((B * (T_NLP + 1), D), jnp.float32),
            jax.ShapeDtypeStruct((NW * TREG, D), jnp.float32),
        ],
        mesh=mesh,
        compiler_params=pltpu.CompilerParams(needs_layout_passes=False),
        scratch_types=[
            pltpu.VMEM((PAIRS_PER_W * KT,), jnp.int32),
            pltpu.VMEM((B * T_NLP,), jnp.int32),
            pltpu.VMEM((B * KEPT_NLP,), jnp.int32),
            pltpu.VMEM((TROWS_PER_W,), jnp.int32),
            pltpu.VMEM((272,), jnp.int32),
            pltpu.VMEM((2, 112, D), jnp.float32),
            pltpu.SemaphoreType.DMA,
            pltpu.SemaphoreType.DMA,
        ],
    )
    t_out, i_out, n_out, _ = f(t_flat, img_flat, nlp_flat, mask_rep, t_idx,
                               img_idx, nlp_idx, img_mask, nlp_mask)
    return t_out, i_out, n_out


def kernel(temporal, img, nlp, temporal_revert_idx, img_revert_idx,
           nlp_revert_idx, img_remain_padding_mask, nlp_remain_padding_mask,
           mask_token):
    t_out, i_out, n_out = _revert_sc(
        temporal.reshape(B * S * 3, D),
        img.reshape(B * KEPT_IMG, D),
        nlp.reshape(B * KEPT_NLP, D),
        jnp.broadcast_to(mask_token, (NMREP, D)),
        temporal_revert_idx.reshape(B * S * KT),
        img_revert_idx.reshape(B * T_IMG),
        nlp_revert_idx.reshape(B * T_NLP),
        img_remain_padding_mask.reshape(B * KEPT_IMG),
        nlp_remain_padding_mask.reshape(B * KEPT_NLP))
    return (t_out.reshape(B, S, KT + 1, D),
            i_out.reshape(B, T_IMG + 1, D),
            n_out.reshape(B, T_NLP + 1, D))


# speedup vs baseline: 6.8742x; 2.8846x over previous
import functools

import jax
import jax.numpy as jnp
from jax import lax
from jax.experimental import pallas as pl
from jax.experimental.pallas import tpu as pltpu
from jax.experimental.pallas import tpu_sc as plsc

B, S, D = 16, 128, 256
M_VALID = 2
KT = 8
T_IMG, KEPT_IMG = 196, 50
T_NLP, KEPT_NLP = 256, 65

ROWS_T = B * S * 3
ROWS_IMG = B * KEPT_IMG
ROWS_NLP = B * KEPT_NLP
BASE_IMG = ROWS_T
BASE_NLP = ROWS_T + ROWS_IMG
MASK_ROW = ROWS_T + ROWS_IMG + ROWS_NLP
NREP = 512

NW = 32
PAIRS_PER_W = (B * S) // NW
TROWS_PER_W = PAIRS_PER_W * (KT + 1)
T_CHUNK = 96


def _iota16():
    return lax.iota(jnp.int32, 16)


def _compute_temporal_gidx(wid, tix, gidxt):
    r0 = wid * PAIRS_PER_W

    def body(c, carry):
        q = c * 16 + _iota16()
        j = lax.shift_right_logical(q * 7282, 16)
        k = q - j * (KT + 1)
        pos = jnp.maximum(j * KT + k - 1, 0)
        raw = plsc.load_gather(tix, [pos])
        gr3 = (r0 + j) * 3
        mrow = MASK_ROW + ((r0 * (KT + 1) + q) & (NREP - 1))
        src = jnp.where(k == 0, gr3,
                        jnp.where(raw < M_VALID, gr3 + 1 + raw, mrow))
        gidxt[pl.ds(c * 16, 16)] = src
        return carry

    lax.fori_loop(0, TROWS_PER_W // 16, body, 0)


def _compute_static_gidx(b, t_len, kept, base, six, mskv, gidxs, didxs,
                         n_chunks):
    ibase = b * t_len
    mbase = b * kept

    def body(c, carry):
        p = c * 16 + _iota16()
        pos = ibase + jnp.clip(p - 1, 0, t_len - 1)
        raw = plsc.load_gather(six, [pos])
        srow = jnp.where(p == 0, 0, 1 + raw)
        mg = jnp.minimum(srow, kept - 1)
        mval = plsc.load_gather(mskv, [mbase + mg])
        valid = (srow <= kept - 1) & (mval == 1)
        mrow = MASK_ROW + ((b * 271 + p * 3) & (NREP - 1))
        src = jnp.where(valid, base + b * kept + srow, mrow)
        gidxs[pl.ds(c * 16, 16)] = src
        didxs[pl.ds(c * 16, 16)] = b * (t_len + 1) + jnp.minimum(p, t_len)
        return carry

    lax.fori_loop(0, n_chunks, body, 0)


def _gather_rows(table, gidx, rows, sem, sem2, out, out_base, chunks,
                 didx=None):
    nc = len(chunks)

    def start_g(i, buf):
        off, n = chunks[i]
        return pltpu.async_copy(
            table.at[gidx.at[pl.ds(off, n)]], rows.at[buf].at[pl.ds(0, n)],
            sem)

    def start_s(i, buf):
        off, n = chunks[i]
        if didx is None:
            dst = out.at[pl.ds(out_base + off, n)]
        else:
            dst = out.at[didx.at[pl.ds(off, n)]]
        return pltpu.async_copy(rows.at[buf].at[pl.ds(0, n)], dst, sem2)

    gd = [None] * nc
    sd = [None] * nc
    gd[0] = start_g(0, 0)
    for i in range(nc):
        gd[i].wait()
        if i + 1 < nc:
            if i >= 1:
                sd[i - 1].wait()
            gd[i + 1] = start_g(i + 1, (i + 1) % 2)
        sd[i] = start_s(i, i % 2)
    if nc >= 2:
        sd[nc - 2].wait()
    sd[nc - 1].wait()


def _sc_body(table, t_idx, img_idx, nlp_idx, img_mask, nlp_mask,
             t_out, img_out, nlp_out, tix, gidxt, six, mskv, gidxs, didxs,
             rows, sem, sem2):
    wid = lax.axis_index("s") * 2 + lax.axis_index("c")

    pltpu.sync_copy(t_idx.at[pl.ds(wid * PAIRS_PER_W * KT, PAIRS_PER_W * KT)],
                    tix)
    _compute_temporal_gidx(wid, tix, gidxt)
    t_chunks = [(h * T_CHUNK, T_CHUNK)
                for h in range(TROWS_PER_W // T_CHUNK)]
    _gather_rows(table, gidxt, rows, sem, sem2, t_out, wid * TROWS_PER_W,
                 t_chunks)

    b = wid // 2
    m = wid - 2 * b

    @pl.when(m == 0)
    def _img():
        pltpu.sync_copy(img_idx, six.at[pl.ds(0, B * T_IMG)])
        pltpu.sync_copy(img_mask, mskv.at[pl.ds(0, B * KEPT_IMG)])
        _compute_static_gidx(b, T_IMG, KEPT_IMG, BASE_IMG, six, mskv, gidxs,
                             didxs, (T_IMG + 1 + 15) // 16)
        _gather_rows(table, gidxs, rows, sem, sem2, img_out, 0,
                     [(0, 112), (112, 88)], didx=didxs)

    @pl.when(m == 1)
    def _nlp():
        pltpu.sync_copy(nlp_idx, six)
        pltpu.sync_copy(nlp_mask, mskv)
        _compute_static_gidx(b, T_NLP, KEPT_NLP, BASE_NLP, six, mskv, gidxs,
                             didxs, (T_NLP + 1 + 15) // 16)
        _gather_rows(table, gidxs, rows, sem, sem2, nlp_out, 0,
                     [(0, 112), (112, 112), (224, 40)], didx=didxs)


@jax.jit
def _revert_sc(table, t_idx, img_idx, nlp_idx, img_mask, nlp_mask):
    mesh = plsc.VectorSubcoreMesh(core_axis_name="c", subcore_axis_name="s")
    f = pl.kernel(
        _sc_body,
        out_type=[
            jax.ShapeDtypeStruct((B * S * (KT + 1), D), jnp.float32),
            jax.ShapeDtypeStruct((B * (T_IMG + 1), D), jnp.float32),
            jax.ShapeDtypeStruct((B * (T_NLP + 1), D), jnp.float32),
        ],
        mesh=mesh,
        compiler_params=pltpu.CompilerParams(needs_layout_passes=False),
        scratch_types=[
            pltpu.VMEM((PAIRS_PER_W * KT,), jnp.int32),
            pltpu.VMEM((TROWS_PER_W,), jnp.int32),
            pltpu.VMEM((B * T_NLP,), jnp.int32),
            pltpu.VMEM((B * KEPT_NLP,), jnp.int32),
            pltpu.VMEM((272,), jnp.int32),
            pltpu.VMEM((272,), jnp.int32),
            pltpu.VMEM((2, 112, D), jnp.float32),
            pltpu.SemaphoreType.DMA,
            pltpu.SemaphoreType.DMA,
        ],
    )
    return f(table, t_idx, img_idx, nlp_idx, img_mask, nlp_mask)


def kernel(temporal, img, nlp, temporal_revert_idx, img_revert_idx,
           nlp_revert_idx, img_remain_padding_mask, nlp_remain_padding_mask,
           mask_token):
    table = jnp.concatenate(
        [temporal.reshape(ROWS_T, D), img.reshape(ROWS_IMG, D),
         nlp.reshape(ROWS_NLP, D),
         jnp.broadcast_to(mask_token, (NREP, D))], axis=0)
    t_idx = temporal_revert_idx.reshape(B * S * KT)
    img_idx = img_revert_idx.reshape(B * T_IMG)
    nlp_idx = nlp_revert_idx.reshape(B * T_NLP)
    img_mask = img_remain_padding_mask.reshape(B * KEPT_IMG)
    nlp_mask = nlp_remain_padding_mask.reshape(B * KEPT_NLP)
    t_out, i_out, n_out = _revert_sc(table, t_idx, img_idx, nlp_idx,
                                     img_mask, nlp_mask)
    return (t_out.reshape(B, S, KT + 1, D),
            i_out.reshape(B, T_IMG + 1, D),
            n_out.reshape(B, T_NLP + 1, D))
